# Initial kernel scaffold; baseline (speedup 1.0000x reference)
import jax
import jax.numpy as jnp
from jax.experimental import pallas as pl


def kernel(node_scalar, node_vector, edge_scalar, edge_vector, frames, params, edge_index):
    raise NotImplementedError("write your pallas kernel here")


# speedup vs baseline: 3.3827x; 3.3827x over previous
import functools

import jax
import jax.numpy as jnp
from jax import lax
from jax.experimental import pallas as pl
from jax.experimental.pallas import tpu as pltpu

N_NODES = 10000
N_EDGES = 160000
E_PAD = 163840
BE = 512

_INTERPRET = False


def _silu(x):
    return x * jax.nn.sigmoid(x)


def _dense_body(nsr, nsc, nvr, nvc, es, ev, fr,
                wvd0, ws0a, ws0b, ws0e, ws0vn, ws0sc, bs0, wvu0, wg0, bg0,
                wvd, wss, wsvn, wssc, bs, wvu, wg, bg, wa, ba,
                msg_s, msg_v):
    f32 = jnp.float32
    dot = functools.partial(jnp.dot, preferred_element_type=f32)

    frb = fr[...]

    vd = []
    for c in range(3):
        vd_c = (dot(nvr[:, 16 * c:16 * c + 16], wvd0[0:16, :])
                + dot(nvc[:, 16 * c:16 * c + 16], wvd0[16:32, :])
                + dot(ev[:, 4 * c:4 * c + 4], wvd0[32:36, :]))
        vd.append(vd_c)
    vn = jnp.sqrt(vd[0] * vd[0] + vd[1] * vd[1] + vd[2] * vd[2] + 1e-8)
    pre = (dot(nsr[...], ws0a[...]) + dot(nsc[...], ws0b[...])
           + dot(es[...], ws0e[...]) + dot(vn, ws0vn[...]))
    for r in range(3):
        sc_r = (vd[0] * frb[:, 3 * r:3 * r + 1]
                + vd[1] * frb[:, 3 * r + 1:3 * r + 2]
                + vd[2] * frb[:, 3 * r + 2:3 * r + 3])
        pre = pre + dot(sc_r, ws0sc[r])
    rs = _silu(pre + bs0[...])
    gate = jax.nn.sigmoid(dot(rs, wg0[...]) + bg0[...])
    rv = [dot(vd[c], wvu0[...]) * gate for c in range(3)]

    for k in range(3):
        vdk = [dot(rv[c], wvd[k]) for c in range(3)]
        vnk = jnp.sqrt(vdk[0] * vdk[0] + vdk[1] * vdk[1] + vdk[2] * vdk[2] + 1e-8)
        prek = dot(rs, wss[k]) + dot(vnk, wsvn[k])
        for r in range(3):
            sc_r = (vdk[0] * frb[:, 3 * r:3 * r + 1]
                    + vdk[1] * frb[:, 3 * r + 1:3 * r + 2]
                    + vdk[2] * frb[:, 3 * r + 2:3 * r + 3])
            prek = prek + dot(sc_r, wssc[k, r])
        ns_k = _silu(prek + bs[k])
        gate = jax.nn.sigmoid(dot(ns_k, wg[k]) + bg[k])
        rs = rs + ns_k
        rv = [rv[c] + dot(vdk[c], wvu[k]) * gate for c in range(3)]

    attn = jax.nn.sigmoid(dot(rs, wa[...]) + ba[...])
    msg_s[...] = rs * attn
    msg_v[...] = jnp.concatenate(rv, axis=-1)


def _run_dense(nsr, nsc, nvr, nvc, es, ev, fr, w):
    grid = E_PAD // BE

    def eb(width):
        return pl.BlockSpec((BE, width), lambda i: (i, 0))

    def full(arr):
        nd = arr.ndim
        return pl.BlockSpec(arr.shape, lambda i, _n=nd: (0,) * _n)

    weights = [w['wvd0'], w['ws0a'], w['ws0b'], w['ws0e'], w['ws0vn'],
               w['ws0sc'], w['bs0'], w['wvu0'], w['wg0'], w['bg0'],
               w['wvd'], w['wss'], w['wsvn'], w['wssc'], w['bs'],
               w['wvu'], w['wg'], w['bg'], w['wa'], w['ba']]

    return pl.pallas_call(
        _dense_body,
        grid=(grid,),
        in_specs=[eb(128), eb(128), eb(48), eb(48), eb(16), eb(12), eb(9)]
                 + [full(x) for x in weights],
        out_specs=[eb(128), eb(48)],
        out_shape=[jax.ShapeDtypeStruct((E_PAD, 128), jnp.float32),
                   jax.ShapeDtypeStruct((E_PAD, 48), jnp.float32)],
        compiler_params=pltpu.CompilerParams(
            dimension_semantics=("arbitrary",)),
        interpret=_INTERPRET,
    )(nsr, nsc, nvr, nvc, es, ev, fr, *weights)


def _prep_weights(params):
    l0 = params['l0']
    ws0 = l0['Ws']
    w = {
        'wvd0': l0['Wvd'],
        'ws0a': ws0[0:128], 'ws0b': ws0[128:256], 'ws0e': ws0[256:272],
        'ws0vn': ws0[272:308],
        'ws0sc': jnp.stack([ws0[308 + r::3] for r in range(3)]),
        'bs0': l0['bs'][None, :],
        'wvu0': l0['Wvu'],
        'wg0': l0['Wg'], 'bg0': l0['bg'][None, :],
        'wa': params['Wa'], 'ba': params['ba'][None, :],
    }
    ls = [params[n] for n in ('l1', 'l2', 'l3')]
    w['wvd'] = jnp.stack([p['Wvd'] for p in ls])
    w['wss'] = jnp.stack([p['Ws'][0:128] for p in ls])
    w['wsvn'] = jnp.stack([p['Ws'][128:144] for p in ls])
    w['wssc'] = jnp.stack([jnp.stack([p['Ws'][144 + r::3] for r in range(3)])
                           for p in ls])
    w['bs'] = jnp.stack([p['bs'][None, :] for p in ls])
    w['wvu'] = jnp.stack([p['Wvu'] for p in ls])
    w['wg'] = jnp.stack([p['Wg'] for p in ls])
    w['bg'] = jnp.stack([p['bg'][None, :] for p in ls])
    return w


def kernel(node_scalar, node_vector, edge_scalar, edge_vector, frames, params,
           edge_index):
    row, col = edge_index[0], edge_index[1]
    pad = E_PAD - N_EDGES
    i32 = jnp.int32

    nv2 = node_vector.transpose(0, 2, 1).reshape(N_NODES, 48)
    ev2 = jnp.pad(edge_vector.transpose(0, 2, 1).reshape(N_EDGES, 12),
                  ((0, pad), (0, 0)))
    fr2 = jnp.pad(frames.reshape(N_EDGES, 9), ((0, pad), (0, 0)))
    es2 = jnp.pad(edge_scalar, ((0, pad), (0, 0)))

    rowp = jnp.concatenate([row, jnp.zeros((pad,), i32)])
    colp = jnp.concatenate([col, jnp.zeros((pad,), i32)])

    nsr = node_scalar[rowp]
    nsc = node_scalar[colp]
    nvr = nv2[rowp]
    nvc = nv2[colp]

    w = _prep_weights(params)
    msg_s, msg_v = _run_dense(nsr, nsc, nvr, nvc, es2, ev2, fr2, w)

    scat = jnp.concatenate([row, jnp.full((pad,), N_NODES, i32)])
    out_s = jax.ops.segment_sum(msg_s, scat, num_segments=N_NODES + 1)[:N_NODES]
    out_v48 = jax.ops.segment_sum(msg_v, scat, num_segments=N_NODES + 1)[:N_NODES]

    out_v = out_v48.reshape(N_NODES, 3, 16).transpose(0, 2, 1)
    return out_s, out_v


# speedup vs baseline: 5.6310x; 1.6646x over previous
import functools

import jax
import jax.numpy as jnp
from jax import lax
from jax.experimental import pallas as pl
from jax.experimental.pallas import tpu as pltpu
from jax.experimental.pallas import tpu_sc as plsc

N_NODES = 10000
N_EDGES = 160000
E_PAD = 163840
BE = 512

NC, NS, LANES = 2, 16, 16
NW = NC * NS
CHUNK = 128
CPW = E_PAD // (NW * CHUNK)
NACC = 10112
ROWS = NACC // NS

_INTERPRET = False

_SC_MESH = plsc.VectorSubcoreMesh(core_axis_name="c", subcore_axis_name="s",
                                  num_cores=NC, num_subcores=NS)


@functools.partial(
    pl.kernel, mesh=_SC_MESH,
    out_type=[jax.ShapeDtypeStruct((E_PAD, 128), jnp.float32),
              jax.ShapeDtypeStruct((E_PAD, 128), jnp.float32),
              jax.ShapeDtypeStruct((E_PAD, 128), jnp.float32),
              jax.ShapeDtypeStruct((E_PAD, 128), jnp.float32)],
    scratch_types=[pltpu.VMEM((CPW, CHUNK), jnp.int32),
                   pltpu.VMEM((CPW, CHUNK), jnp.int32),
                   pltpu.VMEM((CHUNK, 128), jnp.float32),
                   pltpu.VMEM((CHUNK, 128), jnp.float32),
                   pltpu.VMEM((CHUNK, 128), jnp.float32),
                   pltpu.VMEM((CHUNK, 128), jnp.float32),
                   pltpu.SemaphoreType.DMA],
)
def _sc_gather(ns_hbm, nv_hbm, row2d, col2d,
               nsr_hbm, nsc_hbm, nvr_hbm, nvc_hbm,
               rowv, colv, b0, b1, b2, b3, sem):
    wid = lax.axis_index("s") * NC + lax.axis_index("c")
    rbase = wid * CPW
    pltpu.sync_copy(row2d.at[pl.ds(rbase, CPW)], rowv)
    pltpu.sync_copy(col2d.at[pl.ds(rbase, CPW)], colv)

    def body(j, carry):
        ebase = (rbase + j) * CHUNK
        d0 = pltpu.async_copy(ns_hbm.at[rowv.at[j]], b0, sem)
        d1 = pltpu.async_copy(ns_hbm.at[colv.at[j]], b1, sem)
        d2 = pltpu.async_copy(nv_hbm.at[rowv.at[j]], b2, sem)
        d3 = pltpu.async_copy(nv_hbm.at[colv.at[j]], b3, sem)
        d0.wait(); d1.wait(); d2.wait(); d3.wait()
        pltpu.sync_copy(b0, nsr_hbm.at[pl.ds(ebase, CHUNK)])
        pltpu.sync_copy(b1, nsc_hbm.at[pl.ds(ebase, CHUNK)])
        pltpu.sync_copy(b2, nvr_hbm.at[pl.ds(ebase, CHUNK)])
        pltpu.sync_copy(b3, nvc_hbm.at[pl.ds(ebase, CHUNK)])
        return carry

    lax.fori_loop(0, CPW, body, 0)


def _make_scatter(width):
    @functools.partial(
        pl.kernel, mesh=_SC_MESH,
        out_type=jax.ShapeDtypeStruct((NC, NACC, width), jnp.float32),
        scratch_types=[pltpu.VMEM((CPW, CHUNK), jnp.int32),
                       pltpu.VMEM((CHUNK, width), jnp.float32),
                       pltpu.VMEM_SHARED((NACC, width), jnp.float32)],
    )
    def _k(msg_hbm, scat2d, z_hbm, out_hbm, idxv, buf, acc):
        cid = lax.axis_index("c")
        sid = lax.axis_index("s")
        wid = sid * NC + cid
        rbase = wid * CPW

        pltpu.sync_copy(z_hbm.at[pl.ds(sid * ROWS, ROWS)],
                        acc.at[pl.ds(sid * ROWS, ROWS)])
        pltpu.sync_copy(scat2d.at[pl.ds(rbase, CPW)], idxv)
        plsc.subcore_barrier()

        def body(j, carry):
            ebase = (rbase + j) * CHUNK
            pltpu.sync_copy(msg_hbm.at[pl.ds(ebase, CHUNK)], buf)
            pltpu.sync_copy(buf, acc.at[idxv.at[j]], add=True)
            return carry

        lax.fori_loop(0, CPW, body, 0)
        plsc.subcore_barrier()

        pltpu.sync_copy(acc.at[pl.ds(sid * ROWS, ROWS)],
                        out_hbm.at[cid].at[pl.ds(sid * ROWS, ROWS)])

    return _k


_sc_scatter_s = _make_scatter(128)
_sc_scatter_v = _make_scatter(128)


def _combine_body(ps, pv, os_, ov_):
    os_[...] = ps[0] + ps[1]
    ov_[...] = (pv[0] + pv[1])[:, :48]


def _run_combine(ps, pv):
    BN = 400
    return pl.pallas_call(
        _combine_body,
        grid=(N_NODES // BN,),
        in_specs=[pl.BlockSpec((NC, BN, 128), lambda i: (0, i, 0)),
                  pl.BlockSpec((NC, BN, 128), lambda i: (0, i, 0))],
        out_specs=[pl.BlockSpec((BN, 128), lambda i: (i, 0)),
                   pl.BlockSpec((BN, 48), lambda i: (i, 0))],
        out_shape=[jax.ShapeDtypeStruct((N_NODES, 128), jnp.float32),
                   jax.ShapeDtypeStruct((N_NODES, 48), jnp.float32)],
        interpret=_INTERPRET,
    )(ps, pv)


def _silu(x):
    return x * jax.nn.sigmoid(x)


def _dense_body(nsr, nsc, nvr, nvc, es, ev, fr,
                wvd0, ws0a, ws0b, ws0e, ws0vn, ws0sc, bs0, wvu0, wg0, bg0,
                wvd, wss, wsvn, wssc, bs, wvu, wg, bg, wa, ba,
                msg_s, msg_v):
    f32 = jnp.float32
    dot = functools.partial(jnp.dot, preferred_element_type=f32)

    frb = fr[...]

    vd = []
    for c in range(3):
        vd_c = (dot(nvr[:, 16 * c:16 * c + 16], wvd0[0:16, :])
                + dot(nvc[:, 16 * c:16 * c + 16], wvd0[16:32, :])
                + dot(ev[:, 4 * c:4 * c + 4], wvd0[32:36, :]))
        vd.append(vd_c)
    vn = jnp.sqrt(vd[0] * vd[0] + vd[1] * vd[1] + vd[2] * vd[2] + 1e-8)
    pre = (dot(nsr[...], ws0a[...]) + dot(nsc[...], ws0b[...])
           + dot(es[...], ws0e[...]) + dot(vn, ws0vn[...]))
    for r in range(3):
        sc_r = (vd[0] * frb[:, 3 * r:3 * r + 1]
                + vd[1] * frb[:, 3 * r + 1:3 * r + 2]
                + vd[2] * frb[:, 3 * r + 2:3 * r + 3])
        pre = pre + dot(sc_r, ws0sc[r])
    rs = _silu(pre + bs0[...])
    gate = jax.nn.sigmoid(dot(rs, wg0[...]) + bg0[...])
    rv = [dot(vd[c], wvu0[...]) * gate for c in range(3)]

    for k in range(3):
        vdk = [dot(rv[c], wvd[k]) for c in range(3)]
        vnk = jnp.sqrt(vdk[0] * vdk[0] + vdk[1] * vdk[1] + vdk[2] * vdk[2] + 1e-8)
        prek = dot(rs, wss[k]) + dot(vnk, wsvn[k])
        for r in range(3):
            sc_r = (vdk[0] * frb[:, 3 * r:3 * r + 1]
                    + vdk[1] * frb[:, 3 * r + 1:3 * r + 2]
                    + vdk[2] * frb[:, 3 * r + 2:3 * r + 3])
            prek = prek + dot(sc_r, wssc[k, r])
        ns_k = _silu(prek + bs[k])
        gate = jax.nn.sigmoid(dot(ns_k, wg[k]) + bg[k])
        rs = rs + ns_k
        rv = [rv[c] + dot(vdk[c], wvu[k]) * gate for c in range(3)]

    attn = jax.nn.sigmoid(dot(rs, wa[...]) + ba[...])
    msg_s[...] = rs * attn
    msg_v[...] = jnp.concatenate(rv + [jnp.zeros_like(nsr[...], shape=(rv[0].shape[0], 80))],
                                 axis=-1)


def _run_dense(nsr, nsc, nvr, nvc, es, ev, fr, w):
    grid = E_PAD // BE

    def eb(width):
        return pl.BlockSpec((BE, width), lambda i: (i, 0))

    def full(arr):
        nd = arr.ndim
        return pl.BlockSpec(arr.shape, lambda i, _n=nd: (0,) * _n)

    weights = [w['wvd0'], w['ws0a'], w['ws0b'], w['ws0e'], w['ws0vn'],
               w['ws0sc'], w['bs0'], w['wvu0'], w['wg0'], w['bg0'],
               w['wvd'], w['wss'], w['wsvn'], w['wssc'], w['bs'],
               w['wvu'], w['wg'], w['bg'], w['wa'], w['ba']]

    return pl.pallas_call(
        _dense_body,
        grid=(grid,),
        in_specs=[eb(128), eb(128), eb(128), eb(128), eb(16), eb(12), eb(9)]
                 + [full(x) for x in weights],
        out_specs=[eb(128), eb(128)],
        out_shape=[jax.ShapeDtypeStruct((E_PAD, 128), jnp.float32),
                   jax.ShapeDtypeStruct((E_PAD, 128), jnp.float32)],
        compiler_params=pltpu.CompilerParams(
            dimension_semantics=("arbitrary",)),
        interpret=_INTERPRET,
    )(nsr, nsc, nvr, nvc, es, ev, fr, *weights)


def _prep_weights(params):
    l0 = params['l0']
    ws0 = l0['Ws']
    w = {
        'wvd0': l0['Wvd'],
        'ws0a': ws0[0:128], 'ws0b': ws0[128:256], 'ws0e': ws0[256:272],
        'ws0vn': ws0[272:308],
        'ws0sc': jnp.stack([ws0[308 + r::3] for r in range(3)]),
        'bs0': l0['bs'][None, :],
        'wvu0': l0['Wvu'],
        'wg0': l0['Wg'], 'bg0': l0['bg'][None, :],
        'wa': params['Wa'], 'ba': params['ba'][None, :],
    }
    ls = [params[n] for n in ('l1', 'l2', 'l3')]
    w['wvd'] = jnp.stack([p['Wvd'] for p in ls])
    w['wss'] = jnp.stack([p['Ws'][0:128] for p in ls])
    w['wsvn'] = jnp.stack([p['Ws'][128:144] for p in ls])
    w['wssc'] = jnp.stack([jnp.stack([p['Ws'][144 + r::3] for r in range(3)])
                           for p in ls])
    w['bs'] = jnp.stack([p['bs'][None, :] for p in ls])
    w['wvu'] = jnp.stack([p['Wvu'] for p in ls])
    w['wg'] = jnp.stack([p['Wg'] for p in ls])
    w['bg'] = jnp.stack([p['bg'][None, :] for p in ls])
    return w


def kernel(node_scalar, node_vector, edge_scalar, edge_vector, frames, params,
           edge_index):
    row, col = edge_index[0], edge_index[1]
    pad = E_PAD - N_EDGES
    i32 = jnp.int32

    nv2 = jnp.pad(node_vector.transpose(0, 2, 1).reshape(N_NODES, 48),
                  ((0, 0), (0, 80)))
    ev2 = jnp.pad(edge_vector.transpose(0, 2, 1).reshape(N_EDGES, 12),
                  ((0, pad), (0, 0)))
    fr2 = jnp.pad(frames.reshape(N_EDGES, 9), ((0, pad), (0, 0)))
    es2 = jnp.pad(edge_scalar, ((0, pad), (0, 0)))

    row2d = jnp.concatenate([row, jnp.zeros((pad,), i32)]).reshape(-1, CHUNK)
    col2d = jnp.concatenate([col, jnp.zeros((pad,), i32)]).reshape(-1, CHUNK)

    nsr, nsc, nvr, nvc = _sc_gather(node_scalar, nv2, row2d, col2d)

    w = _prep_weights(params)
    msg_s, msg_v = _run_dense(nsr, nsc, nvr, nvc, es2, ev2, fr2, w)

    scat2d = jnp.concatenate([row, jnp.full((pad,), N_NODES, i32)]).reshape(-1, CHUNK)
    zs = jnp.zeros((NACC, 128), jnp.float32)
    zv = jnp.zeros((NACC, 128), jnp.float32)
    part_s = _sc_scatter_s(msg_s, scat2d, zs)
    part_v = _sc_scatter_v(msg_v, scat2d, zv)
    out_s, out_v48 = _run_combine(part_s, part_v)

    out_v = out_v48.reshape(N_NODES, 3, 16).transpose(0, 2, 1)
    return out_s, out_v
